# baseline (speedup 1.0000x reference)
import functools

import jax
import jax.numpy as jnp
from jax import lax
from jax.experimental import pallas as pl
from jax.experimental.pallas import tpu as pltpu
from jax.experimental.pallas import tpu_sc as plsc

_EPS = 1e-15

_N = 10000
_D = 128
_E = 320000
_NW = 32
_PER_W = (2 * _E) // _NW
_CHUNK = 80
_NCHUNK = _PER_W // _CHUNK


def _edge_partials_sc(z, src_idx, dst_idx):
    mesh = plsc.VectorSubcoreMesh(core_axis_name="c", subcore_axis_name="s")

    @functools.partial(
        pl.kernel,
        mesh=mesh,
        out_type=jax.ShapeDtypeStruct((2 * _E, 16), jnp.float32),
        scratch_types=[
            pltpu.VMEM((_PER_W,), jnp.int32),
            pltpu.VMEM((_PER_W,), jnp.int32),
            pltpu.VMEM((_CHUNK, _D), jnp.float32),
            pltpu.VMEM((_CHUNK, _D), jnp.float32),
            pltpu.VMEM((_CHUNK, _D), jnp.float32),
            pltpu.VMEM((_CHUNK, _D), jnp.float32),
            pltpu.VMEM((_CHUNK, 16), jnp.float32),
            pltpu.VMEM((_CHUNK, 16), jnp.float32),
            pltpu.SemaphoreType.DMA,
            pltpu.SemaphoreType.DMA,
            pltpu.SemaphoreType.DMA,
            pltpu.SemaphoreType.DMA,
        ],
    )
    def sck(z_hbm, si_hbm, di_hbm, out_hbm,
            si_v, di_v, sa, da, sb, db, oa, ob, semA, semB, semOA, semOB):
        wid = lax.axis_index("s") * 2 + lax.axis_index("c")
        base_w = wid * _PER_W

        pltpu.sync_copy(si_hbm.at[pl.ds(base_w, _PER_W)], si_v)
        pltpu.sync_copy(di_hbm.at[pl.ds(base_w, _PER_W)], di_v)

        def issue(c, sbuf, dbuf, sem):
            pltpu.async_copy(z_hbm.at[si_v.at[pl.ds(c * _CHUNK, _CHUNK)]],
                             sbuf, sem)
            pltpu.async_copy(z_hbm.at[di_v.at[pl.ds(c * _CHUNK, _CHUNK)]],
                             dbuf, sem)

        def wait(sbuf, dbuf, sem):
            pltpu.make_async_copy(z_hbm.at[si_v.at[pl.ds(0, _CHUNK)]],
                                  sbuf, sem).wait()
            pltpu.make_async_copy(z_hbm.at[di_v.at[pl.ds(0, _CHUNK)]],
                                  dbuf, sem).wait()

        def compute(srows, drows, obuf):
            for r in range(_CHUNK):
                acc = srows[r, pl.ds(0, 16)] * drows[r, pl.ds(0, 16)]
                for kk in range(1, _D // 16):
                    acc = acc + (srows[r, pl.ds(kk * 16, 16)]
                                 * drows[r, pl.ds(kk * 16, 16)])
                obuf[r, :] = acc

        def issue_out(c, obuf, sem):
            pltpu.async_copy(
                obuf, out_hbm.at[pl.ds(base_w + c * _CHUNK, _CHUNK)], sem)

        def wait_out(obuf, sem):
            pltpu.make_async_copy(
                obuf, out_hbm.at[pl.ds(base_w, _CHUNK)], sem).wait()

        issue(0, sa, da, semA)

        def pair_body(p, carry):
            c0 = 2 * p
            issue(c0 + 1, sb, db, semB)
            wait(sa, da, semA)

            @pl.when(p > 0)
            def _():
                wait_out(oa, semOA)

            compute(sa, da, oa)
            issue_out(c0, oa, semOA)

            @pl.when(p < _NCHUNK // 2 - 1)
            def _():
                issue(c0 + 2, sa, da, semA)

            wait(sb, db, semB)

            @pl.when(p > 0)
            def _():
                wait_out(ob, semOB)

            compute(sb, db, ob)
            issue_out(c0 + 1, ob, semOB)
            return carry

        lax.fori_loop(0, _NCHUNK // 2, pair_body, 0)
        wait_out(oa, semOA)
        wait_out(ob, semOB)

    return sck(z, src_idx, dst_idx)


_BLOCKS = 20
_BROWS = (2 * _E * 16 // 128) // _BLOCKS


def _bce_loss_tc(parts):
    def body(x_ref, o_ref):
        pid = pl.program_id(0)

        @pl.when(pid == 0)
        def _():
            o_ref[...] = jnp.zeros((1, 1), jnp.float32)

        x = x_ref[...]
        jidx = lax.broadcasted_iota(jnp.int32, (_D, 8), 0)
        gidx = lax.broadcasted_iota(jnp.int32, (_D, 8), 1)
        fold = (jidx // 16 == gidx).astype(jnp.float32)
        v = lax.dot_general(x, fold, (((1,), (0,)), ((), ())),
                            preferred_element_type=jnp.float32)
        sig = jax.nn.sigmoid(v)
        lp = jnp.sum(jnp.log(sig + _EPS))
        ln = jnp.sum(jnp.log(1.0 - sig + _EPS))
        term = jnp.where(pid < _BLOCKS // 2, lp, ln)
        o_ref[...] += -term.reshape(1, 1) / _E

    out = pl.pallas_call(
        body,
        grid=(_BLOCKS,),
        in_specs=[pl.BlockSpec((_BROWS, _D), lambda i: (i, 0))],
        out_specs=pl.BlockSpec((1, 1), lambda i: (0, 0)),
        out_shape=jax.ShapeDtypeStruct((1, 1), jnp.float32),
    )(parts)
    return out.reshape(())


def kernel(z, pos_edge_index, neg_edge_index):
    src = jnp.concatenate(
        [pos_edge_index[0], neg_edge_index[0]]).astype(jnp.int32)
    dst = jnp.concatenate(
        [pos_edge_index[1], neg_edge_index[1]]).astype(jnp.int32)
    parts = _edge_partials_sc(z, src, dst)
    return _bce_loss_tc(parts.reshape(2 * _E * 16 // _D, _D))


# speedup vs baseline: 1.1872x; 1.1872x over previous
import functools

import jax
import jax.numpy as jnp
from jax import lax
from jax.experimental import pallas as pl
from jax.experimental.pallas import tpu as pltpu
from jax.experimental.pallas import tpu_sc as plsc

_EPS = 1e-15

_N = 10000
_D = 128
_E = 320000
_NW = 32
_PER_W = (2 * _E) // _NW
_CHUNK = 80
_NCHUNK = _PER_W // _CHUNK


def _edge_partials_sc(z, src_idx, dst_idx):
    mesh = plsc.VectorSubcoreMesh(core_axis_name="c", subcore_axis_name="s")

    @functools.partial(
        pl.kernel,
        mesh=mesh,
        out_type=jax.ShapeDtypeStruct((2 * _E * 16,), jnp.float32),
        scratch_types=[
            pltpu.VMEM((_PER_W,), jnp.int32),
            pltpu.VMEM((_PER_W,), jnp.int32),
            pltpu.VMEM((_CHUNK, _D), jnp.float32),
            pltpu.VMEM((_CHUNK, _D), jnp.float32),
            pltpu.VMEM((_CHUNK, _D), jnp.float32),
            pltpu.VMEM((_CHUNK, _D), jnp.float32),
            pltpu.VMEM((_CHUNK * 16,), jnp.float32),
            pltpu.VMEM((_CHUNK * 16,), jnp.float32),
            pltpu.SemaphoreType.DMA,
            pltpu.SemaphoreType.DMA,
            pltpu.SemaphoreType.DMA,
            pltpu.SemaphoreType.DMA,
        ],
    )
    def sck(z_hbm, si_hbm, di_hbm, out_hbm,
            si_v, di_v, sa, da, sb, db, oa, ob, semA, semB, semOA, semOB):
        wid = lax.axis_index("s") * 2 + lax.axis_index("c")
        base_w = wid * _PER_W

        pltpu.sync_copy(si_hbm.at[pl.ds(base_w, _PER_W)], si_v)
        pltpu.sync_copy(di_hbm.at[pl.ds(base_w, _PER_W)], di_v)

        def issue(c, sbuf, dbuf, sem):
            pltpu.async_copy(z_hbm.at[si_v.at[pl.ds(c * _CHUNK, _CHUNK)]],
                             sbuf, sem)
            pltpu.async_copy(z_hbm.at[di_v.at[pl.ds(c * _CHUNK, _CHUNK)]],
                             dbuf, sem)

        def wait(sbuf, dbuf, sem):
            pltpu.make_async_copy(z_hbm.at[si_v.at[pl.ds(0, _CHUNK)]],
                                  sbuf, sem).wait()
            pltpu.make_async_copy(z_hbm.at[di_v.at[pl.ds(0, _CHUNK)]],
                                  dbuf, sem).wait()

        def compute(srows, drows, obuf):
            for r in range(_CHUNK):
                acc = srows[r, pl.ds(0, 16)] * drows[r, pl.ds(0, 16)]
                for kk in range(1, _D // 16):
                    acc = acc + (srows[r, pl.ds(kk * 16, 16)]
                                 * drows[r, pl.ds(kk * 16, 16)])
                obuf[pl.ds(r * 16, 16)] = acc

        def issue_out(c, obuf, sem):
            pltpu.async_copy(
                obuf,
                out_hbm.at[pl.ds((base_w + c * _CHUNK) * 16, _CHUNK * 16)],
                sem)

        def wait_out(obuf, sem):
            pltpu.make_async_copy(
                obuf, out_hbm.at[pl.ds(base_w * 16, _CHUNK * 16)], sem).wait()

        issue(0, sa, da, semA)

        def pair_body(p, carry):
            c0 = 2 * p
            issue(c0 + 1, sb, db, semB)
            wait(sa, da, semA)

            @pl.when(p > 0)
            def _():
                wait_out(oa, semOA)

            compute(sa, da, oa)
            issue_out(c0, oa, semOA)

            @pl.when(p < _NCHUNK // 2 - 1)
            def _():
                issue(c0 + 2, sa, da, semA)

            wait(sb, db, semB)

            @pl.when(p > 0)
            def _():
                wait_out(ob, semOB)

            compute(sb, db, ob)
            issue_out(c0 + 1, ob, semOB)
            return carry

        lax.fori_loop(0, _NCHUNK // 2, pair_body, 0)
        wait_out(oa, semOA)
        wait_out(ob, semOB)

    return sck(z, src_idx, dst_idx)


_BLOCKS = 20
_BROWS = (2 * _E * 16 // 128) // _BLOCKS


def _bce_loss_tc(parts):
    def body(x_ref, o_ref):
        pid = pl.program_id(0)

        @pl.when(pid == 0)
        def _():
            o_ref[...] = jnp.zeros((1, 1), jnp.float32)

        x = x_ref[...]
        jidx = lax.broadcasted_iota(jnp.int32, (_D, 8), 0)
        gidx = lax.broadcasted_iota(jnp.int32, (_D, 8), 1)
        fold = (jidx // 16 == gidx).astype(jnp.float32)
        v = lax.dot_general(x, fold, (((1,), (0,)), ((), ())),
                            preferred_element_type=jnp.float32)
        sig = jax.nn.sigmoid(v)
        lp = jnp.sum(jnp.log(sig + _EPS))
        ln = jnp.sum(jnp.log(1.0 - sig + _EPS))
        term = jnp.where(pid < _BLOCKS // 2, lp, ln)
        o_ref[...] += -term.reshape(1, 1) / _E

    out = pl.pallas_call(
        body,
        grid=(_BLOCKS,),
        in_specs=[pl.BlockSpec((_BROWS, _D), lambda i: (i, 0))],
        out_specs=pl.BlockSpec((1, 1), lambda i: (0, 0)),
        out_shape=jax.ShapeDtypeStruct((1, 1), jnp.float32),
    )(parts)
    return out.reshape(())


def kernel(z, pos_edge_index, neg_edge_index):
    src = jnp.concatenate(
        [pos_edge_index[0], neg_edge_index[0]]).astype(jnp.int32)
    dst = jnp.concatenate(
        [pos_edge_index[1], neg_edge_index[1]]).astype(jnp.int32)
    parts = _edge_partials_sc(z, src, dst)
    return _bce_loss_tc(parts.reshape(2 * _E * 16 // _D, _D))
